# baseline (speedup 1.0000x reference)
import jax
import jax.numpy as jnp
from jax.experimental import pallas as pl
from jax.experimental.pallas import tpu as pltpu

ROWS = 16384
T = 49
F0 = 48
NCHUNK = 4
CBLK = ROWS // NCHUNK


def _body(x_hbm, o_ref, *scratch):
    bufs = scratch[:NCHUNK]
    sems = scratch[NCHUNK:]
    copies = []
    for n in range(NCHUNK):
        cp = pltpu.async_copy(
            x_hbm.at[T, pl.ds(F0, 16), pl.ds(n * CBLK, CBLK)],
            bufs[n],
            sems[n],
            priority=n % 2,
        )
        copies.append(cp)
    for n in range(NCHUNK):
        copies[n].wait()
        v = bufs[n]
        s = jnp.sum(v[0:8], axis=0) + v[8] + v[9]
        o_ref[pl.ds(n * CBLK, CBLK)] = jnp.where(
            s > 0, jnp.ones_like(s), jnp.zeros_like(s)
        )


@jax.jit
def kernel(x):
    xt = jnp.transpose(x, (1, 2, 0))
    out = pl.pallas_call(
        _body,
        in_specs=[pl.BlockSpec(memory_space=pl.ANY)],
        out_specs=pl.BlockSpec((ROWS,), lambda: (0,)),
        out_shape=jax.ShapeDtypeStruct((ROWS,), jnp.float32),
        scratch_shapes=(
            [pltpu.VMEM((16, CBLK), jnp.float32) for _ in range(NCHUNK)]
            + [pltpu.SemaphoreType.DMA for _ in range(NCHUNK)]
        ),
    )(xt)
    return out.reshape(ROWS, 1)


# speedup vs baseline: 1.0206x; 1.0206x over previous
import jax
import jax.numpy as jnp
from jax.experimental import pallas as pl
from jax.experimental.pallas import tpu as pltpu

ROWS = 16384
T = 49
F0 = 48


def _body(x_hbm, o_ref, buf_a, buf_b, sem_a, sem_b):
    cp_a = pltpu.make_async_copy(x_hbm.at[T, pl.ds(F0, 8), :], buf_a, sem_a)
    cp_b = pltpu.make_async_copy(x_hbm.at[T, pl.ds(F0 + 8, 8), :], buf_b, sem_b)
    cp_a.start()
    cp_b.start()
    cp_a.wait()
    s8 = jnp.sum(buf_a[...], axis=0)
    cp_b.wait()
    s = s8 + buf_b[0] + buf_b[1]
    o_ref[...] = jnp.where(s > 0, jnp.ones_like(s), jnp.zeros_like(s))


@jax.jit
def kernel(x):
    xt = jnp.transpose(x, (1, 2, 0))
    out = pl.pallas_call(
        _body,
        in_specs=[pl.BlockSpec(memory_space=pl.ANY)],
        out_specs=pl.BlockSpec((ROWS,), lambda: (0,)),
        out_shape=jax.ShapeDtypeStruct((ROWS,), jnp.float32),
        scratch_shapes=[
            pltpu.VMEM((8, ROWS), jnp.float32),
            pltpu.VMEM((8, ROWS), jnp.float32),
            pltpu.SemaphoreType.DMA,
            pltpu.SemaphoreType.DMA,
        ],
    )(xt)
    return out.reshape(ROWS, 1)


# speedup vs baseline: 1.0272x; 1.0064x over previous
import jax
import jax.numpy as jnp
from jax.experimental import pallas as pl
from jax.experimental.pallas import tpu as pltpu

ROWS = 16384
T = 49
F0 = 48
NCHUNK = 4
CBLK = ROWS // NCHUNK


def _body(x_hbm, o_ref, *scratch):
    bufs = scratch[:NCHUNK]
    sems = scratch[NCHUNK:]
    copies = []
    for n in range(NCHUNK):
        cp = pltpu.make_async_copy(
            x_hbm.at[T, pl.ds(F0, 16), pl.ds(n * CBLK, CBLK)],
            bufs[n],
            sems[n],
        )
        cp.start()
        copies.append(cp)
    for n in range(NCHUNK):
        copies[n].wait()
        v = bufs[n]
        s = jnp.sum(v[0:8], axis=0) + v[8] + v[9]
        o_ref[pl.ds(n * CBLK, CBLK)] = jnp.where(
            s > 0, jnp.ones_like(s), jnp.zeros_like(s)
        )


@jax.jit
def kernel(x):
    xt = jnp.transpose(x, (1, 2, 0))
    out = pl.pallas_call(
        _body,
        in_specs=[pl.BlockSpec(memory_space=pl.ANY)],
        out_specs=pl.BlockSpec((ROWS,), lambda: (0,)),
        out_shape=jax.ShapeDtypeStruct((ROWS,), jnp.float32),
        scratch_shapes=(
            [pltpu.VMEM((16, CBLK), jnp.float32) for _ in range(NCHUNK)]
            + [pltpu.SemaphoreType.DMA for _ in range(NCHUNK)]
        ),
    )(xt)
    return out.reshape(ROWS, 1)
